# baseline (speedup 1.0000x reference)
import dataclasses
import functools

import jax
import jax.numpy as jnp
from jax import lax
from jax.experimental import pallas as pl
from jax.experimental.pallas import tpu as pltpu
from jax.experimental.pallas import tpu_sc as plsc

N = 10000
E = 320000
F_IN = 128
H = 128
C = 16

NC = 2
NS = 16
NW = NC * NS
LANES = 16
CHUNK = 128
NCHUNKS = 80
PW = NCHUNKS * CHUNK
EPAD = NW * PW
NPAD = 10240
WCH = 128
ZCH = NPAD // WCH // NS

RB = 1000
GRID = N // RB


def _sc_mesh():
    return plsc.VectorSubcoreMesh(core_axis_name="c", subcore_axis_name="s")


def _sc_params():
    cp = pltpu.CompilerParams()
    if "needs_layout_passes" in pltpu.CompilerParams.__dataclass_fields__:
        cp = dataclasses.replace(cp, needs_layout_passes=False)
    return cp


def _deg_kernel(dst_hbm, w_hbm, out_hbm,
                w_all, dstA, dstB, bufA, bufB, acc, lsemA, lsemB, ssemA, ssemB):
    cid = lax.axis_index("c")
    sid = lax.axis_index("s")
    wid = cid * NS + sid

    pltpu.sync_copy(w_hbm.at[wid], w_all)

    @pl.loop(0, CHUNK)
    def _zero_buf(j):
        bufA[j, pl.ds(0, LANES)] = jnp.zeros((LANES,), jnp.float32)

    @pl.loop(0, ZCH)
    def _zero_acc(i):
        r = (sid * ZCH + i) * WCH
        pltpu.sync_copy(bufA, acc.at[pl.ds(r, WCH)])

    plsc.subcore_barrier()

    def _fill(buf, g):
        @pl.loop(0, CHUNK)
        def _f(j):
            wb = plsc.load_gather(w_all, [jnp.full((LANES,), 0, jnp.int32) + (g * CHUNK + j)])
            buf[j, pl.ds(0, LANES)] = wb

    @pl.loop(0, NCHUNKS, step=2)
    def _edges(g):
        dA = pltpu.async_copy(dst_hbm.at[wid].at[pl.ds(g * CHUNK, CHUNK)], dstA, lsemA)
        dB = pltpu.async_copy(dst_hbm.at[wid].at[pl.ds((g + 1) * CHUNK, CHUNK)], dstB, lsemB)
        _fill(bufA, g)
        dA.wait()
        h1 = pltpu.async_copy(bufA, acc.at[dstA], ssemA, add=True)
        _fill(bufB, g + 1)
        h1.wait()
        dB.wait()
        h2 = pltpu.async_copy(bufB, acc.at[dstB], ssemB, add=True)
        h2.wait()

    plsc.subcore_barrier()

    @pl.loop(0, ZCH)
    def _writeback(i):
        r = (sid * ZCH + i) * WCH
        pltpu.sync_copy(acc.at[pl.ds(r, WCH)], out_hbm.at[cid].at[pl.ds(r, WCH)])


def _sc_degree(dstp, wp):
    k = functools.partial(
        pl.kernel,
        mesh=_sc_mesh(),
        compiler_params=_sc_params(),
        out_type=jax.ShapeDtypeStruct((NC, NPAD, LANES), jnp.float32),
        scratch_types=[
            pltpu.VMEM((PW,), jnp.float32),
            pltpu.VMEM((CHUNK,), jnp.int32),
            pltpu.VMEM((CHUNK,), jnp.int32),
            pltpu.VMEM((CHUNK, LANES), jnp.float32),
            pltpu.VMEM((CHUNK, LANES), jnp.float32),
            pltpu.VMEM_SHARED((NPAD, LANES), jnp.float32),
            pltpu.SemaphoreType.DMA,
            pltpu.SemaphoreType.DMA,
            pltpu.SemaphoreType.DMA,
            pltpu.SemaphoreType.DMA,
        ],
    )(_deg_kernel)
    return k(dstp, wp)


def _msg_kernel(src_hbm, dst_hbm, w_hbm, h_hbm, out_hbm,
                srcA, srcB, dstA, dstB, wvA, wvB, rowsA, rowsB, acc,
                lsemA, lsemB, gsemA, gsemB):
    cid = lax.axis_index("c")
    sid = lax.axis_index("s")
    wid = cid * NS + sid

    @pl.loop(0, CHUNK)
    def _zero_rows(j):
        for kk in range(H // LANES):
            rowsA[j, pl.ds(kk * LANES, LANES)] = jnp.zeros((LANES,), jnp.float32)

    @pl.loop(0, ZCH)
    def _zero_acc(i):
        r = (sid * ZCH + i) * WCH
        pltpu.sync_copy(rowsA, acc.at[pl.ds(r, WCH)])

    plsc.subcore_barrier()

    def _scale(wv, buf):
        @pl.loop(0, CHUNK, step=4)
        def _s(j):
            wbs = [plsc.load_gather(wv, [jnp.full((LANES,), 0, jnp.int32) + (j + r)])
                   for r in range(4)]
            for kk in range(H // LANES):
                sl = pl.ds(kk * LANES, LANES)
                for r in range(4):
                    buf[j + r, sl] = buf[j + r, sl] * wbs[r]

    @pl.loop(0, NCHUNKS, step=2)
    def _edges(g):
        @pl.when(g > 0)
        def _():
            pltpu.make_async_copy(rowsB, acc.at[dstB], gsemB).wait()

        ca = pl.ds(g * CHUNK, CHUNK)
        cb = pl.ds((g + 1) * CHUNK, CHUNK)
        sA = pltpu.async_copy(src_hbm.at[wid].at[ca], srcA, lsemA)
        dA = pltpu.async_copy(dst_hbm.at[wid].at[ca], dstA, lsemA)
        wA = pltpu.async_copy(w_hbm.at[wid].at[ca], wvA, lsemA)
        sB = pltpu.async_copy(src_hbm.at[wid].at[cb], srcB, lsemB)
        dB = pltpu.async_copy(dst_hbm.at[wid].at[cb], dstB, lsemB)
        wB = pltpu.async_copy(w_hbm.at[wid].at[cb], wvB, lsemB)
        sA.wait()
        gA = pltpu.async_copy(h_hbm.at[srcA], rowsA, gsemA)
        sB.wait()
        gB = pltpu.async_copy(h_hbm.at[srcB], rowsB, gsemB)
        wA.wait()
        gA.wait()
        _scale(wvA, rowsA)
        dA.wait()
        hA = pltpu.async_copy(rowsA, acc.at[dstA], gsemA, add=True)
        wB.wait()
        gB.wait()
        _scale(wvB, rowsB)
        hA.wait()
        dB.wait()
        pltpu.async_copy(rowsB, acc.at[dstB], gsemB, add=True)

    pltpu.make_async_copy(rowsB, acc.at[dstB], gsemB).wait()
    plsc.subcore_barrier()

    @pl.loop(0, ZCH)
    def _writeback(i):
        r = (sid * ZCH + i) * WCH
        pltpu.sync_copy(acc.at[pl.ds(r, WCH)], out_hbm.at[cid].at[pl.ds(r, WCH)])


def _sc_aggregate(srcp, dstp, wp, hprime):
    k = functools.partial(
        pl.kernel,
        mesh=_sc_mesh(),
        compiler_params=_sc_params(),
        out_type=jax.ShapeDtypeStruct((NC, NPAD, H), jnp.float32),
        scratch_types=[
            pltpu.VMEM((CHUNK,), jnp.int32),
            pltpu.VMEM((CHUNK,), jnp.int32),
            pltpu.VMEM((CHUNK,), jnp.int32),
            pltpu.VMEM((CHUNK,), jnp.int32),
            pltpu.VMEM((CHUNK,), jnp.float32),
            pltpu.VMEM((CHUNK,), jnp.float32),
            pltpu.VMEM((CHUNK, H), jnp.float32),
            pltpu.VMEM((CHUNK, H), jnp.float32),
            pltpu.VMEM_SHARED((NPAD, H), jnp.float32),
            pltpu.SemaphoreType.DMA,
            pltpu.SemaphoreType.DMA,
            pltpu.SemaphoreType.DMA,
            pltpu.SemaphoreType.DMA,
        ],
    )(_msg_kernel)
    return k(srcp, dstp, wp, hprime)


def _dinv_from_parts(degp):
    deg = degp[0, :, 0:1] + degp[1, :, 0:1] + 1.0
    return lax.rsqrt(deg)


def _pre1_body(xb, wt1, degp, hp_out):
    dinv = _dinv_from_parts(degp[...])
    h = jnp.dot(xb[...], wt1[...], preferred_element_type=jnp.float32)
    hp_out[...] = h * dinv


def _post_body(accp, hp, degp, b, wtn, out_l, hp_next):
    dinv = _dinv_from_parts(degp[...])
    a = accp[...]
    g = (a[0] + a[1] + hp[...]) * dinv + b[...]
    nrm = jnp.sqrt(jnp.sum(g * g, axis=1, keepdims=True))
    g = g / jnp.maximum(nrm, 1e-12)
    o = jnp.maximum(g, 0.0)
    out_l[...] = o
    hp_next[...] = jnp.dot(o, wtn[...], preferred_element_type=jnp.float32) * dinv


def _final_body(accp, hp, degp, b, o1, o2, wlt, bl, y):
    dinv = _dinv_from_parts(degp[...])
    a = accp[...]
    g = (a[0] + a[1] + hp[...]) * dinv + b[...]
    nrm = jnp.sqrt(jnp.sum(g * g, axis=1, keepdims=True))
    g = g / jnp.maximum(nrm, 1e-12)
    o3 = jnp.maximum(g, 0.0)
    w = wlt[...]
    y[...] = (
        jnp.dot(o1[...], w[0:H], preferred_element_type=jnp.float32)
        + jnp.dot(o2[...], w[H:2 * H], preferred_element_type=jnp.float32)
        + jnp.dot(o3, w[2 * H:3 * H], preferred_element_type=jnp.float32)
        + bl[...]
    )


def _row_spec(width):
    return pl.BlockSpec((RB, width), lambda i: (i, 0))


def _parts_spec(width):
    return pl.BlockSpec((NC, RB, width), lambda i: (0, i, 0))


def _full_spec(shape):
    return pl.BlockSpec(shape, lambda i: tuple(0 for _ in shape))


def _tc_pre1(x, wt1, degp):
    return pl.pallas_call(
        _pre1_body,
        grid=(GRID,),
        in_specs=[_row_spec(F_IN), _full_spec((F_IN, H)), _parts_spec(LANES)],
        out_specs=_row_spec(H),
        out_shape=jax.ShapeDtypeStruct((N, H), jnp.float32),
    )(x, wt1, degp)


def _tc_post(accp, hp, degp, b, wtn):
    return pl.pallas_call(
        _post_body,
        grid=(GRID,),
        in_specs=[
            _parts_spec(H),
            _row_spec(H),
            _parts_spec(LANES),
            _full_spec((1, H)),
            _full_spec((H, H)),
        ],
        out_specs=[_row_spec(H), _row_spec(H)],
        out_shape=[
            jax.ShapeDtypeStruct((N, H), jnp.float32),
            jax.ShapeDtypeStruct((N, H), jnp.float32),
        ],
    )(accp, hp, degp, b, wtn)


def _tc_final(accp, hp, degp, b, o1, o2, wlt, bl):
    return pl.pallas_call(
        _final_body,
        grid=(GRID,),
        in_specs=[
            _parts_spec(H),
            _row_spec(H),
            _parts_spec(LANES),
            _full_spec((1, H)),
            _row_spec(H),
            _row_spec(H),
            _full_spec((3 * H, C)),
            _full_spec((1, C)),
        ],
        out_specs=_row_spec(C),
        out_shape=jax.ShapeDtypeStruct((N, C), jnp.float32),
    )(accp, hp, degp, b, o1, o2, wlt, bl)


def kernel(x, edge_index, edge_weights, W1, b1, W2, b2, W3, b3, Wl, bl):
    src = edge_index[0]
    dst = edge_index[1]
    pad = EPAD - E
    zi = jnp.arange(pad, dtype=jnp.int32) % N
    srcp = jnp.concatenate([src, zi]).reshape(NW, PW)
    dstp = jnp.concatenate([dst, zi]).reshape(NW, PW)
    wp = jnp.concatenate([edge_weights, jnp.zeros((pad,), jnp.float32)]).reshape(NW, PW)

    degp = _sc_degree(dstp, wp)

    hp1 = _tc_pre1(x, W1.T, degp)
    acc1 = _sc_aggregate(srcp, dstp, wp, hp1)
    out1, hp2 = _tc_post(acc1, hp1, degp, b1.reshape(1, H), W2.T)
    acc2 = _sc_aggregate(srcp, dstp, wp, hp2)
    out2, hp3 = _tc_post(acc2, hp2, degp, b2.reshape(1, H), W3.T)
    acc3 = _sc_aggregate(srcp, dstp, wp, hp3)
    y = _tc_final(acc3, hp3, degp, b3.reshape(1, H), out1, out2, Wl.T, bl.reshape(1, C))
    return y


# speedup vs baseline: 1.0886x; 1.0886x over previous
import dataclasses
import functools

import jax
import jax.numpy as jnp
from jax import lax
from jax.experimental import pallas as pl
from jax.experimental.pallas import tpu as pltpu
from jax.experimental.pallas import tpu_sc as plsc

N = 10000
E = 320000
F_IN = 128
H = 128
C = 16

NC = 2
NS = 16
NW = NC * NS
LANES = 16
CHUNK = 128
NCHUNKS = 80
PW = NCHUNKS * CHUNK
EPAD = NW * PW
NPAD = 10240
WCH = 128
ZCH = NPAD // WCH // NS

RB = 1000
GRID = N // RB


def _sc_mesh():
    return plsc.VectorSubcoreMesh(core_axis_name="c", subcore_axis_name="s")


def _sc_params():
    cp = pltpu.CompilerParams()
    if "needs_layout_passes" in pltpu.CompilerParams.__dataclass_fields__:
        cp = dataclasses.replace(cp, needs_layout_passes=False)
    return cp


def _deg_kernel(dst_hbm, w_hbm, out_hbm,
                w_all, dstA, dstB, bufA, bufB, acc, lsemA, lsemB, ssemA, ssemB):
    cid = lax.axis_index("c")
    sid = lax.axis_index("s")
    wid = cid * NS + sid

    pltpu.sync_copy(w_hbm.at[wid], w_all)

    @pl.loop(0, CHUNK)
    def _zero_buf(j):
        bufA[j, pl.ds(0, LANES)] = jnp.zeros((LANES,), jnp.float32)

    @pl.loop(0, ZCH)
    def _zero_acc(i):
        r = (sid * ZCH + i) * WCH
        pltpu.sync_copy(bufA, acc.at[pl.ds(r, WCH)])

    plsc.subcore_barrier()

    def _fill(buf, g):
        @pl.loop(0, CHUNK)
        def _f(j):
            wb = plsc.load_gather(w_all, [jnp.full((LANES,), 0, jnp.int32) + (g * CHUNK + j)])
            buf[j, pl.ds(0, LANES)] = wb

    @pl.loop(0, NCHUNKS, step=2)
    def _edges(g):
        dA = pltpu.async_copy(dst_hbm.at[wid].at[pl.ds(g * CHUNK, CHUNK)], dstA, lsemA)
        dB = pltpu.async_copy(dst_hbm.at[wid].at[pl.ds((g + 1) * CHUNK, CHUNK)], dstB, lsemB)
        _fill(bufA, g)
        dA.wait()
        h1 = pltpu.async_copy(bufA, acc.at[dstA], ssemA, add=True)
        _fill(bufB, g + 1)
        h1.wait()
        dB.wait()
        h2 = pltpu.async_copy(bufB, acc.at[dstB], ssemB, add=True)
        h2.wait()

    plsc.subcore_barrier()

    @pl.loop(0, ZCH)
    def _writeback(i):
        r = (sid * ZCH + i) * WCH
        pltpu.sync_copy(acc.at[pl.ds(r, WCH)], out_hbm.at[cid].at[pl.ds(r, WCH)])


def _sc_degree(dstp, wp):
    k = functools.partial(
        pl.kernel,
        mesh=_sc_mesh(),
        compiler_params=_sc_params(),
        out_type=jax.ShapeDtypeStruct((NC, NPAD, LANES), jnp.float32),
        scratch_types=[
            pltpu.VMEM((PW,), jnp.float32),
            pltpu.VMEM((CHUNK,), jnp.int32),
            pltpu.VMEM((CHUNK,), jnp.int32),
            pltpu.VMEM((CHUNK, LANES), jnp.float32),
            pltpu.VMEM((CHUNK, LANES), jnp.float32),
            pltpu.VMEM_SHARED((NPAD, LANES), jnp.float32),
            pltpu.SemaphoreType.DMA,
            pltpu.SemaphoreType.DMA,
            pltpu.SemaphoreType.DMA,
            pltpu.SemaphoreType.DMA,
        ],
    )(_deg_kernel)
    return k(dstp, wp)


def _msg_kernel(src_hbm, dst_hbm, w_hbm, h_hbm, out_hbm,
                src0, src1, wv0, wv1, dst0, dst1, dst2, dst3, rows0, rows1, acc,
                lsem0, lsem1, gsem0, gsem1, ssem0, ssem1):
    cid = lax.axis_index("c")
    sid = lax.axis_index("s")
    wid = cid * NS + sid

    srcs = (src0, src1)
    wvs = (wv0, wv1)
    dsts = (dst0, dst1, dst2, dst3)
    rows = (rows0, rows1)
    lsems = (lsem0, lsem1)
    gsems = (gsem0, gsem1)
    ssems = (ssem0, ssem1)

    @pl.loop(0, CHUNK)
    def _zero_rows(j):
        for kk in range(H // LANES):
            rows0[j, pl.ds(kk * LANES, LANES)] = jnp.zeros((LANES,), jnp.float32)

    @pl.loop(0, ZCH)
    def _zero_acc(i):
        r = (sid * ZCH + i) * WCH
        pltpu.sync_copy(rows0, acc.at[pl.ds(r, WCH)])

    plsc.subcore_barrier()

    def _scale(wv, buf):
        @plsc.parallel_loop(0, CHUNK, unroll=4)
        def _s(j):
            wb = plsc.load_gather(wv, [jnp.full((LANES,), j, jnp.int32)])
            for kk in range(H // LANES):
                sl = pl.ds(kk * LANES, LANES)
                buf[j, sl] = buf[j, sl] * wb

    def _loads_start(c, i2, i4):
        cs = pl.ds(c * CHUNK, CHUNK)
        pltpu.async_copy(src_hbm.at[wid].at[cs], srcs[i2], lsems[i2])
        pltpu.async_copy(w_hbm.at[wid].at[cs], wvs[i2], lsems[i2])
        pltpu.async_copy(dst_hbm.at[wid].at[cs], dsts[i4], lsems[i2])

    def _loads_wait(c, i2, i4):
        cs = pl.ds(c * CHUNK, CHUNK)
        pltpu.make_async_copy(src_hbm.at[wid].at[cs], srcs[i2], lsems[i2]).wait()
        pltpu.make_async_copy(w_hbm.at[wid].at[cs], wvs[i2], lsems[i2]).wait()
        pltpu.make_async_copy(dst_hbm.at[wid].at[cs], dsts[i4], lsems[i2]).wait()

    def _gather_start(i2):
        pltpu.async_copy(h_hbm.at[srcs[i2]], rows[i2], gsems[i2])

    def _gather_wait(i2):
        pltpu.make_async_copy(h_hbm.at[srcs[i2]], rows[i2], gsems[i2]).wait()

    def _scatter_start(i2, i4):
        pltpu.async_copy(rows[i2], acc.at[dsts[i4]], ssems[i2], add=True)

    def _scatter_wait(i2, i4):
        pltpu.make_async_copy(rows[i2], acc.at[dsts[i4]], ssems[i2]).wait()

    _loads_start(0, 0, 0)
    _loads_start(1, 1, 1)
    _loads_wait(0, 0, 0)
    _gather_start(0)

    @pl.loop(0, NCHUNKS, step=4)
    def _edges(g):
        for k in range(4):
            c = g + k
            i2 = k % 2
            o2 = (k + 1) % 2
            _gather_wait(i2)
            _scale(wvs[i2], rows[i2])

            if k == 0:
                @pl.when(g > 0)
                def _():
                    _scatter_wait(o2, 3)
            else:
                _scatter_wait(o2, k - 1)

            _scatter_start(i2, k)

            @pl.when(c + 2 < NCHUNKS)
            def _():
                _loads_start(c + 2, i2, (k + 2) % 4)

            @pl.when(c + 1 < NCHUNKS)
            def _():
                _loads_wait(c + 1, o2, (k + 1) % 4)
                _gather_start(o2)

    _scatter_wait(1, 3)
    plsc.subcore_barrier()

    @pl.loop(0, ZCH)
    def _writeback(i):
        r = (sid * ZCH + i) * WCH
        pltpu.sync_copy(acc.at[pl.ds(r, WCH)], out_hbm.at[cid].at[pl.ds(r, WCH)])


def _sc_aggregate(srcp, dstp, wp, hprime):
    k = functools.partial(
        pl.kernel,
        mesh=_sc_mesh(),
        compiler_params=_sc_params(),
        out_type=jax.ShapeDtypeStruct((NC, NPAD, H), jnp.float32),
        scratch_types=[
            pltpu.VMEM((CHUNK,), jnp.int32),
            pltpu.VMEM((CHUNK,), jnp.int32),
            pltpu.VMEM((CHUNK,), jnp.float32),
            pltpu.VMEM((CHUNK,), jnp.float32),
            pltpu.VMEM((CHUNK,), jnp.int32),
            pltpu.VMEM((CHUNK,), jnp.int32),
            pltpu.VMEM((CHUNK,), jnp.int32),
            pltpu.VMEM((CHUNK,), jnp.int32),
            pltpu.VMEM((CHUNK, H), jnp.float32),
            pltpu.VMEM((CHUNK, H), jnp.float32),
            pltpu.VMEM_SHARED((NPAD, H), jnp.float32),
            pltpu.SemaphoreType.DMA,
            pltpu.SemaphoreType.DMA,
            pltpu.SemaphoreType.DMA,
            pltpu.SemaphoreType.DMA,
            pltpu.SemaphoreType.DMA,
            pltpu.SemaphoreType.DMA,
        ],
    )(_msg_kernel)
    return k(srcp, dstp, wp, hprime)


def _dinv_from_parts(degp):
    deg = degp[0, :, 0:1] + degp[1, :, 0:1] + 1.0
    return lax.rsqrt(deg)


def _pre1_body(xb, wt1, degp, hp_out):
    dinv = _dinv_from_parts(degp[...])
    h = jnp.dot(xb[...], wt1[...], preferred_element_type=jnp.float32)
    hp_out[...] = h * dinv


def _post_body(accp, hp, degp, b, wtn, out_l, hp_next):
    dinv = _dinv_from_parts(degp[...])
    a = accp[...]
    g = (a[0] + a[1] + hp[...]) * dinv + b[...]
    nrm = jnp.sqrt(jnp.sum(g * g, axis=1, keepdims=True))
    g = g / jnp.maximum(nrm, 1e-12)
    o = jnp.maximum(g, 0.0)
    out_l[...] = o
    hp_next[...] = jnp.dot(o, wtn[...], preferred_element_type=jnp.float32) * dinv


def _final_body(accp, hp, degp, b, o1, o2, wlt, bl, y):
    dinv = _dinv_from_parts(degp[...])
    a = accp[...]
    g = (a[0] + a[1] + hp[...]) * dinv + b[...]
    nrm = jnp.sqrt(jnp.sum(g * g, axis=1, keepdims=True))
    g = g / jnp.maximum(nrm, 1e-12)
    o3 = jnp.maximum(g, 0.0)
    w = wlt[...]
    y[...] = (
        jnp.dot(o1[...], w[0:H], preferred_element_type=jnp.float32)
        + jnp.dot(o2[...], w[H:2 * H], preferred_element_type=jnp.float32)
        + jnp.dot(o3, w[2 * H:3 * H], preferred_element_type=jnp.float32)
        + bl[...]
    )


def _row_spec(width):
    return pl.BlockSpec((RB, width), lambda i: (i, 0))


def _parts_spec(width):
    return pl.BlockSpec((NC, RB, width), lambda i: (0, i, 0))


def _full_spec(shape):
    return pl.BlockSpec(shape, lambda i: tuple(0 for _ in shape))


def _tc_pre1(x, wt1, degp):
    return pl.pallas_call(
        _pre1_body,
        grid=(GRID,),
        in_specs=[_row_spec(F_IN), _full_spec((F_IN, H)), _parts_spec(LANES)],
        out_specs=_row_spec(H),
        out_shape=jax.ShapeDtypeStruct((N, H), jnp.float32),
    )(x, wt1, degp)


def _tc_post(accp, hp, degp, b, wtn):
    return pl.pallas_call(
        _post_body,
        grid=(GRID,),
        in_specs=[
            _parts_spec(H),
            _row_spec(H),
            _parts_spec(LANES),
            _full_spec((1, H)),
            _full_spec((H, H)),
        ],
        out_specs=[_row_spec(H), _row_spec(H)],
        out_shape=[
            jax.ShapeDtypeStruct((N, H), jnp.float32),
            jax.ShapeDtypeStruct((N, H), jnp.float32),
        ],
    )(accp, hp, degp, b, wtn)


def _tc_final(accp, hp, degp, b, o1, o2, wlt, bl):
    return pl.pallas_call(
        _final_body,
        grid=(GRID,),
        in_specs=[
            _parts_spec(H),
            _row_spec(H),
            _parts_spec(LANES),
            _full_spec((1, H)),
            _row_spec(H),
            _row_spec(H),
            _full_spec((3 * H, C)),
            _full_spec((1, C)),
        ],
        out_specs=_row_spec(C),
        out_shape=jax.ShapeDtypeStruct((N, C), jnp.float32),
    )(accp, hp, degp, b, o1, o2, wlt, bl)


def kernel(x, edge_index, edge_weights, W1, b1, W2, b2, W3, b3, Wl, bl):
    src = edge_index[0]
    dst = edge_index[1]
    pad = EPAD - E
    zi = jnp.arange(pad, dtype=jnp.int32) % N
    srcp = jnp.concatenate([src, zi]).reshape(NW, PW)
    dstp = jnp.concatenate([dst, zi]).reshape(NW, PW)
    wp = jnp.concatenate([edge_weights, jnp.zeros((pad,), jnp.float32)]).reshape(NW, PW)

    degp = _sc_degree(dstp, wp)

    hp1 = _tc_pre1(x, W1.T, degp)
    acc1 = _sc_aggregate(srcp, dstp, wp, hp1)
    out1, hp2 = _tc_post(acc1, hp1, degp, b1.reshape(1, H), W2.T)
    acc2 = _sc_aggregate(srcp, dstp, wp, hp2)
    out2, hp3 = _tc_post(acc2, hp2, degp, b2.reshape(1, H), W3.T)
    acc3 = _sc_aggregate(srcp, dstp, wp, hp3)
    y = _tc_final(acc3, hp3, degp, b3.reshape(1, H), out1, out2, Wl.T, bl.reshape(1, C))
    return y


# speedup vs baseline: 1.0981x; 1.0087x over previous
import dataclasses
import functools

import jax
import jax.numpy as jnp
from jax import lax
from jax.experimental import pallas as pl
from jax.experimental.pallas import tpu as pltpu
from jax.experimental.pallas import tpu_sc as plsc

N = 10000
E = 320000
F_IN = 128
H = 128
C = 16

NC = 2
NS = 16
NW = NC * NS
LANES = 16
CHUNK = 128
NCHUNKS = 80
PW = NCHUNKS * CHUNK
EPAD = NW * PW
NPAD = 10240
WCH = 128
ZCH = NPAD // WCH // NS

RB = 1000
GRID = N // RB


def _sc_mesh():
    return plsc.VectorSubcoreMesh(core_axis_name="c", subcore_axis_name="s")


def _sc_params():
    cp = pltpu.CompilerParams()
    if "needs_layout_passes" in pltpu.CompilerParams.__dataclass_fields__:
        cp = dataclasses.replace(cp, needs_layout_passes=False)
    return cp


def _deg_kernel(dst_hbm, w_hbm, out_hbm,
                w_all, dst0, dst1, dst2, dst3, bufA, bufB, acc,
                lsem0, lsem1, ssemA, ssemB):
    cid = lax.axis_index("c")
    sid = lax.axis_index("s")
    wid = cid * NS + sid

    dsts = (dst0, dst1, dst2, dst3)
    bufs = (bufA, bufB)
    lsems = (lsem0, lsem1)
    ssems = (ssemA, ssemB)

    pltpu.sync_copy(w_hbm.at[wid], w_all)

    @pl.loop(0, CHUNK)
    def _zero_buf(j):
        bufA[j, pl.ds(0, LANES)] = jnp.zeros((LANES,), jnp.float32)

    @pl.loop(0, ZCH)
    def _zero_acc(i):
        r = (sid * ZCH + i) * WCH
        pltpu.sync_copy(bufA, acc.at[pl.ds(r, WCH)])

    plsc.subcore_barrier()

    def _fill(buf, g):
        @pl.loop(0, CHUNK)
        def _f(j):
            wb = plsc.load_gather(w_all, [jnp.full((LANES,), 0, jnp.int32) + (g * CHUNK + j)])
            buf[j, pl.ds(0, LANES)] = wb

    def _load_start(c, i4):
        pltpu.async_copy(dst_hbm.at[wid].at[pl.ds(c * CHUNK, CHUNK)], dsts[i4], lsems[i4 % 2])

    def _load_wait(c, i4):
        pltpu.make_async_copy(
            dst_hbm.at[wid].at[pl.ds(c * CHUNK, CHUNK)], dsts[i4], lsems[i4 % 2]).wait()

    def _scatter_start(i2, i4):
        pltpu.async_copy(bufs[i2], acc.at[dsts[i4]], ssems[i2], add=True)

    def _scatter_wait(i2, i4):
        pltpu.make_async_copy(bufs[i2], acc.at[dsts[i4]], ssems[i2]).wait()

    _load_start(0, 0)
    _load_start(1, 1)

    @pl.loop(0, NCHUNKS, step=4)
    def _edges(g):
        for k in range(4):
            c = g + k
            i2 = k % 2
            o2 = (k + 1) % 2
            _fill(bufs[i2], c)

            if k == 0:
                @pl.when(g > 0)
                def _():
                    _scatter_wait(o2, 3)
            else:
                _scatter_wait(o2, k - 1)

            _load_wait(c, k)
            _scatter_start(i2, k)

            @pl.when(c + 2 < NCHUNKS)
            def _():
                _load_start(c + 2, (k + 2) % 4)

    _scatter_wait(1, 3)
    plsc.subcore_barrier()

    @pl.loop(0, ZCH)
    def _writeback(i):
        r = (sid * ZCH + i) * WCH
        pltpu.sync_copy(acc.at[pl.ds(r, WCH)], out_hbm.at[cid].at[pl.ds(r, WCH)])


def _sc_degree(dstp, wp):
    k = functools.partial(
        pl.kernel,
        mesh=_sc_mesh(),
        compiler_params=_sc_params(),
        out_type=jax.ShapeDtypeStruct((NC, NPAD, LANES), jnp.float32),
        scratch_types=[
            pltpu.VMEM((PW,), jnp.float32),
            pltpu.VMEM((CHUNK,), jnp.int32),
            pltpu.VMEM((CHUNK,), jnp.int32),
            pltpu.VMEM((CHUNK,), jnp.int32),
            pltpu.VMEM((CHUNK,), jnp.int32),
            pltpu.VMEM((CHUNK, LANES), jnp.float32),
            pltpu.VMEM((CHUNK, LANES), jnp.float32),
            pltpu.VMEM_SHARED((NPAD, LANES), jnp.float32),
            pltpu.SemaphoreType.DMA,
            pltpu.SemaphoreType.DMA,
            pltpu.SemaphoreType.DMA,
            pltpu.SemaphoreType.DMA,
        ],
    )(_deg_kernel)
    return k(dstp, wp)


def _msg_kernel(src_hbm, dst_hbm, w_hbm, h_hbm, out_hbm,
                src0, src1, wv0, wv1, dst0, dst1, dst2, dst3, rows0, rows1, acc,
                lsem0, lsem1, gsem0, gsem1, ssem0, ssem1):
    cid = lax.axis_index("c")
    sid = lax.axis_index("s")
    wid = cid * NS + sid

    srcs = (src0, src1)
    wvs = (wv0, wv1)
    dsts = (dst0, dst1, dst2, dst3)
    rows = (rows0, rows1)
    lsems = (lsem0, lsem1)
    gsems = (gsem0, gsem1)
    ssems = (ssem0, ssem1)

    @pl.loop(0, CHUNK)
    def _zero_rows(j):
        for kk in range(H // LANES):
            rows0[j, pl.ds(kk * LANES, LANES)] = jnp.zeros((LANES,), jnp.float32)

    @pl.loop(0, ZCH)
    def _zero_acc(i):
        r = (sid * ZCH + i) * WCH
        pltpu.sync_copy(rows0, acc.at[pl.ds(r, WCH)])

    plsc.subcore_barrier()

    def _scale(wv, buf):
        @plsc.parallel_loop(0, CHUNK, unroll=4)
        def _s(j):
            wb = plsc.load_gather(wv, [jnp.full((LANES,), j, jnp.int32)])
            for kk in range(H // LANES):
                sl = pl.ds(kk * LANES, LANES)
                buf[j, sl] = buf[j, sl] * wb

    def _loads_start(c, i2, i4):
        cs = pl.ds(c * CHUNK, CHUNK)
        pltpu.async_copy(src_hbm.at[wid].at[cs], srcs[i2], lsems[i2])
        pltpu.async_copy(w_hbm.at[wid].at[cs], wvs[i2], lsems[i2])
        pltpu.async_copy(dst_hbm.at[wid].at[cs], dsts[i4], lsems[i2])

    def _loads_wait(c, i2, i4):
        cs = pl.ds(c * CHUNK, CHUNK)
        pltpu.make_async_copy(src_hbm.at[wid].at[cs], srcs[i2], lsems[i2]).wait()
        pltpu.make_async_copy(w_hbm.at[wid].at[cs], wvs[i2], lsems[i2]).wait()
        pltpu.make_async_copy(dst_hbm.at[wid].at[cs], dsts[i4], lsems[i2]).wait()

    def _gather_start(i2):
        pltpu.async_copy(h_hbm.at[srcs[i2]], rows[i2], gsems[i2])

    def _gather_wait(i2):
        pltpu.make_async_copy(h_hbm.at[srcs[i2]], rows[i2], gsems[i2]).wait()

    def _scatter_start(i2, i4):
        pltpu.async_copy(rows[i2], acc.at[dsts[i4]], ssems[i2], add=True)

    def _scatter_wait(i2, i4):
        pltpu.make_async_copy(rows[i2], acc.at[dsts[i4]], ssems[i2]).wait()

    _loads_start(0, 0, 0)
    _loads_start(1, 1, 1)
    _loads_wait(0, 0, 0)
    _gather_start(0)

    @pl.loop(0, NCHUNKS, step=4)
    def _edges(g):
        for k in range(4):
            c = g + k
            i2 = k % 2
            o2 = (k + 1) % 2
            _gather_wait(i2)
            _scale(wvs[i2], rows[i2])

            if k == 0:
                @pl.when(g > 0)
                def _():
                    _scatter_wait(o2, 3)
            else:
                _scatter_wait(o2, k - 1)

            _scatter_start(i2, k)

            @pl.when(c + 2 < NCHUNKS)
            def _():
                _loads_start(c + 2, i2, (k + 2) % 4)

            @pl.when(c + 1 < NCHUNKS)
            def _():
                _loads_wait(c + 1, o2, (k + 1) % 4)
                _gather_start(o2)

    _scatter_wait(1, 3)
    plsc.subcore_barrier()

    @pl.loop(0, ZCH)
    def _writeback(i):
        r = (sid * ZCH + i) * WCH
        pltpu.sync_copy(acc.at[pl.ds(r, WCH)], out_hbm.at[cid].at[pl.ds(r, WCH)])


def _sc_aggregate(srcp, dstp, wp, hprime):
    k = functools.partial(
        pl.kernel,
        mesh=_sc_mesh(),
        compiler_params=_sc_params(),
        out_type=jax.ShapeDtypeStruct((NC, NPAD, H), jnp.float32),
        scratch_types=[
            pltpu.VMEM((CHUNK,), jnp.int32),
            pltpu.VMEM((CHUNK,), jnp.int32),
            pltpu.VMEM((CHUNK,), jnp.float32),
            pltpu.VMEM((CHUNK,), jnp.float32),
            pltpu.VMEM((CHUNK,), jnp.int32),
            pltpu.VMEM((CHUNK,), jnp.int32),
            pltpu.VMEM((CHUNK,), jnp.int32),
            pltpu.VMEM((CHUNK,), jnp.int32),
            pltpu.VMEM((CHUNK, H), jnp.float32),
            pltpu.VMEM((CHUNK, H), jnp.float32),
            pltpu.VMEM_SHARED((NPAD, H), jnp.float32),
            pltpu.SemaphoreType.DMA,
            pltpu.SemaphoreType.DMA,
            pltpu.SemaphoreType.DMA,
            pltpu.SemaphoreType.DMA,
            pltpu.SemaphoreType.DMA,
            pltpu.SemaphoreType.DMA,
        ],
    )(_msg_kernel)
    return k(srcp, dstp, wp, hprime)


def _dinv_from_parts(degp):
    deg = degp[0, :, 0:1] + degp[1, :, 0:1] + 1.0
    return lax.rsqrt(deg)


def _pre1_body(xb, wt1, degp, hp_out):
    dinv = _dinv_from_parts(degp[...])
    h = jnp.dot(xb[...], wt1[...], preferred_element_type=jnp.float32)
    hp_out[...] = h * dinv


def _post_body(accp, hp, degp, b, wtn, out_l, hp_next):
    dinv = _dinv_from_parts(degp[...])
    a = accp[...]
    g = (a[0] + a[1] + hp[...]) * dinv + b[...]
    nrm = jnp.sqrt(jnp.sum(g * g, axis=1, keepdims=True))
    g = g / jnp.maximum(nrm, 1e-12)
    o = jnp.maximum(g, 0.0)
    out_l[...] = o
    hp_next[...] = jnp.dot(o, wtn[...], preferred_element_type=jnp.float32) * dinv


def _final_body(accp, hp, degp, b, o1, o2, wlt, bl, y):
    dinv = _dinv_from_parts(degp[...])
    a = accp[...]
    g = (a[0] + a[1] + hp[...]) * dinv + b[...]
    nrm = jnp.sqrt(jnp.sum(g * g, axis=1, keepdims=True))
    g = g / jnp.maximum(nrm, 1e-12)
    o3 = jnp.maximum(g, 0.0)
    w = wlt[...]
    y[...] = (
        jnp.dot(o1[...], w[0:H], preferred_element_type=jnp.float32)
        + jnp.dot(o2[...], w[H:2 * H], preferred_element_type=jnp.float32)
        + jnp.dot(o3, w[2 * H:3 * H], preferred_element_type=jnp.float32)
        + bl[...]
    )


def _row_spec(width):
    return pl.BlockSpec((RB, width), lambda i: (i, 0))


def _parts_spec(width):
    return pl.BlockSpec((NC, RB, width), lambda i: (0, i, 0))


def _full_spec(shape):
    return pl.BlockSpec(shape, lambda i: tuple(0 for _ in shape))


def _tc_pre1(x, wt1, degp):
    return pl.pallas_call(
        _pre1_body,
        grid=(GRID,),
        in_specs=[_row_spec(F_IN), _full_spec((F_IN, H)), _parts_spec(LANES)],
        out_specs=_row_spec(H),
        out_shape=jax.ShapeDtypeStruct((N, H), jnp.float32),
    )(x, wt1, degp)


def _tc_post(accp, hp, degp, b, wtn):
    return pl.pallas_call(
        _post_body,
        grid=(GRID,),
        in_specs=[
            _parts_spec(H),
            _row_spec(H),
            _parts_spec(LANES),
            _full_spec((1, H)),
            _full_spec((H, H)),
        ],
        out_specs=[_row_spec(H), _row_spec(H)],
        out_shape=[
            jax.ShapeDtypeStruct((N, H), jnp.float32),
            jax.ShapeDtypeStruct((N, H), jnp.float32),
        ],
    )(accp, hp, degp, b, wtn)


def _tc_final(accp, hp, degp, b, o1, o2, wlt, bl):
    return pl.pallas_call(
        _final_body,
        grid=(GRID,),
        in_specs=[
            _parts_spec(H),
            _row_spec(H),
            _parts_spec(LANES),
            _full_spec((1, H)),
            _row_spec(H),
            _row_spec(H),
            _full_spec((3 * H, C)),
            _full_spec((1, C)),
        ],
        out_specs=_row_spec(C),
        out_shape=jax.ShapeDtypeStruct((N, C), jnp.float32),
    )(accp, hp, degp, b, o1, o2, wlt, bl)


def kernel(x, edge_index, edge_weights, W1, b1, W2, b2, W3, b3, Wl, bl):
    src = edge_index[0]
    dst = edge_index[1]
    pad = EPAD - E
    zi = jnp.arange(pad, dtype=jnp.int32) % N
    srcp = jnp.concatenate([src, zi]).reshape(NW, PW)
    dstp = jnp.concatenate([dst, zi]).reshape(NW, PW)
    wp = jnp.concatenate([edge_weights, jnp.zeros((pad,), jnp.float32)]).reshape(NW, PW)

    degp = _sc_degree(dstp, wp)

    hp1 = _tc_pre1(x, W1.T, degp)
    acc1 = _sc_aggregate(srcp, dstp, wp, hp1)
    out1, hp2 = _tc_post(acc1, hp1, degp, b1.reshape(1, H), W2.T)
    acc2 = _sc_aggregate(srcp, dstp, wp, hp2)
    out2, hp3 = _tc_post(acc2, hp2, degp, b2.reshape(1, H), W3.T)
    acc3 = _sc_aggregate(srcp, dstp, wp, hp3)
    y = _tc_final(acc3, hp3, degp, b3.reshape(1, H), out1, out2, Wl.T, bl.reshape(1, C))
    return y
